# Initial kernel scaffold; baseline (speedup 1.0000x reference)
import jax
import jax.numpy as jnp
from jax.experimental import pallas as pl


def kernel(obs, action, W, b, gamma, beta, keys1, values1, keys2, values2):
    raise NotImplementedError("write your pallas kernel here")


# speedup vs baseline: 1.2124x; 1.2124x over previous
import functools

import jax
import jax.numpy as jnp
from jax.experimental import pallas as pl

K_NEIGHBORS = 32
DELTA = 1e-3
BIG = 3.4e38


def _trunk_body(x_ref, w_ref, b_ref, g_ref, beta_ref, o_ref):
    h = jnp.dot(x_ref[...], w_ref[...], preferred_element_type=jnp.float32)
    h = h + b_ref[...]
    mu = jnp.mean(h, axis=1, keepdims=True)
    var = jnp.mean((h - mu) ** 2, axis=1, keepdims=True)
    hn = (h - mu) / jnp.sqrt(var + 1e-5) * g_ref[...] + beta_ref[...]
    o_ref[...] = jnp.tanh(hn)


def _dist_body(phi_ref, k_ref, o_ref):
    phi = phi_ref[...]
    kb = k_ref[...]
    dot = jax.lax.dot_general(phi, kb, (((1,), (1,)), ((), ())),
                              preferred_element_type=jnp.float32)
    pn = jnp.sum(phi * phi, axis=1, keepdims=True)
    kn = jnp.sum(kb * kb, axis=1)[None, :]
    o_ref[...] = jnp.maximum(pn - 2.0 * dot + kn, 0.0)


def _select_body(d_ref, v_ref, o_ref, *, tr, cap, ch):
    nch = cap // ch
    K = float(K_NEIGHBORS)

    def sweep(_, t):
        def chunk(c, acc):
            m, cgt = acc
            dc = d_ref[:, pl.ds(c * ch, ch)]
            gt = dc > t
            cand = jnp.where(gt, dc, BIG)
            m = jnp.minimum(m, jnp.min(cand, axis=1, keepdims=True))
            cgt = cgt + jnp.sum(gt.astype(jnp.float32), axis=1, keepdims=True)
            return (m, cgt)
        m, cgt = jax.lax.fori_loop(
            0, nch, chunk,
            (jnp.full((tr, 1), BIG, jnp.float32),
             jnp.zeros((tr, 1), jnp.float32)))
        cle = float(cap) - cgt
        return jnp.where(cle >= K, t, m)

    t32 = jax.lax.fori_loop(0, K_NEIGHBORS, sweep,
                            jnp.full((tr, 1), -1.0, jnp.float32))

    def chunk2(c, acc):
        num, den, clt, ceq, sall, j1, v1 = acc
        dc = d_ref[:, pl.ds(c * ch, ch)]
        vb = v_ref[:, pl.ds(c * ch, ch)]
        lt = dc < t32
        eq = dc == t32
        w = jnp.where(lt, 1.0 / (dc + DELTA), 0.0)
        num = num + jnp.sum(w * vb, axis=1, keepdims=True)
        den = den + jnp.sum(w, axis=1, keepdims=True)
        clt = clt + jnp.sum(lt.astype(jnp.float32), axis=1, keepdims=True)
        ceq = ceq + jnp.sum(eq.astype(jnp.float32), axis=1, keepdims=True)
        sall = sall + jnp.sum(jnp.where(eq, vb, 0.0), axis=1, keepdims=True)
        j = (jax.lax.broadcasted_iota(jnp.int32, (tr, ch), 1)
             .astype(jnp.float32) + jnp.float32(c * ch))
        jm = jnp.min(jnp.where(eq, j, BIG), axis=1, keepdims=True)
        vm = jnp.sum(jnp.where(eq & (j == jm), vb, 0.0), axis=1,
                     keepdims=True)
        take = jm < j1
        j1 = jnp.where(take, jm, j1)
        v1 = jnp.where(take, vm, v1)
        return (num, den, clt, ceq, sall, j1, v1)

    z = jnp.zeros((tr, 1), jnp.float32)
    big = jnp.full((tr, 1), BIG, jnp.float32)
    num, den, clt, ceq, sall, j1, v1 = jax.lax.fori_loop(
        0, nch, chunk2, (z, z, z, z, z, big, z))

    need = K - clt
    wstar = 1.0 / (t32 + DELTA)
    tie_num = jnp.where(ceq == need, sall,
                        jnp.where(need == 1.0, v1,
                                  sall * need / jnp.maximum(ceq, 1.0)))
    q = (num + wstar * tie_num) / (den + wstar * need)
    o_ref[...] = jnp.broadcast_to(q, (tr, 128))


def kernel(obs, action, W, b, gamma, beta, keys1, values1, keys2, values2):
    B = obs.shape[0]
    HID = W.shape[1]
    CAP = keys1.shape[0]

    x = jnp.concatenate([obs, action], axis=-1)

    phi = pl.pallas_call(
        _trunk_body,
        out_shape=jax.ShapeDtypeStruct((B, HID), jnp.float32),
    )(x, W, b.reshape(1, HID), gamma.reshape(1, HID), beta.reshape(1, HID))

    BR = min(512, B)
    CB = min(1024, CAP)
    d = pl.pallas_call(
        _dist_body,
        grid=(B // BR, CAP // CB),
        in_specs=[
            pl.BlockSpec((BR, HID), lambda r, c: (r, 0)),
            pl.BlockSpec((CB, HID), lambda r, c: (c, 0)),
        ],
        out_specs=pl.BlockSpec((BR, CB), lambda r, c: (r, c)),
        out_shape=jax.ShapeDtypeStruct((B, CAP), jnp.float32),
    )(phi, keys1)

    TR = min(32, B)
    CH = min(2048, CAP)
    vrow = values1.reshape(1, CAP)
    q = pl.pallas_call(
        functools.partial(_select_body, tr=TR, cap=CAP, ch=CH),
        grid=(B // TR,),
        in_specs=[
            pl.BlockSpec((TR, CAP), lambda r: (r, 0)),
            pl.BlockSpec((1, CAP), lambda r: (0, 0)),
        ],
        out_specs=pl.BlockSpec((TR, 128), lambda r: (r, 0)),
        out_shape=jax.ShapeDtypeStruct((B, 128), jnp.float32),
    )(d, vrow)

    q = q[:, :1]
    return (q, q)


# speedup vs baseline: 1.6954x; 1.3983x over previous
import functools

import jax
import jax.numpy as jnp
from jax.experimental import pallas as pl

K_NEIGHBORS = 32
DELTA = 1e-3
BIG = 3.4e38


def _trunk_body(x_ref, w_ref, b_ref, g_ref, beta_ref, o_ref):
    h = jnp.dot(x_ref[...], w_ref[...], preferred_element_type=jnp.float32)
    h = h + b_ref[...]
    mu = jnp.mean(h, axis=1, keepdims=True)
    var = jnp.mean((h - mu) ** 2, axis=1, keepdims=True)
    hn = (h - mu) / jnp.sqrt(var + 1e-5) * g_ref[...] + beta_ref[...]
    o_ref[...] = jnp.tanh(hn)


def _dist_body(phi_ref, k_ref, o_ref):
    phi = phi_ref[...]
    kb = k_ref[...]
    dot = jax.lax.dot_general(phi, kb, (((1,), (1,)), ((), ())),
                              preferred_element_type=jnp.float32)
    pn = jnp.sum(phi * phi, axis=1, keepdims=True)
    kn = jnp.sum(kb * kb, axis=1)[None, :]
    o_ref[...] = jnp.maximum(pn - 2.0 * dot + kn, 0.0)


def _select_body(d_ref, v_ref, o_ref, *, tr, cap, ch):
    nch = cap // ch
    K = float(K_NEIGHBORS)

    def sweep(_, t):
        def chunk(c, acc):
            m, cgt = acc
            dc = d_ref[:, pl.ds(c * ch, ch)]
            gt = dc > t
            cand = jnp.where(gt, dc, BIG)
            m = jnp.minimum(m, jnp.min(cand, axis=1, keepdims=True))
            cgt = cgt + jnp.sum(gt.astype(jnp.float32), axis=1, keepdims=True)
            return (m, cgt)
        m, cgt = jax.lax.fori_loop(
            0, nch, chunk,
            (jnp.full((tr, 1), BIG, jnp.float32),
             jnp.zeros((tr, 1), jnp.float32)))
        cle = float(cap) - cgt
        return jnp.where(cle >= K, t, m)

    t32 = jax.lax.fori_loop(0, K_NEIGHBORS, sweep,
                            jnp.full((tr, 1), -1.0, jnp.float32))

    def chunk2(c, acc):
        num, den, clt, ceq, sall, j1, v1 = acc
        dc = d_ref[:, pl.ds(c * ch, ch)]
        vb = v_ref[:, pl.ds(c * ch, ch)]
        lt = dc < t32
        eq = dc == t32
        w = jnp.where(lt, 1.0 / (dc + DELTA), 0.0)
        num = num + jnp.sum(w * vb, axis=1, keepdims=True)
        den = den + jnp.sum(w, axis=1, keepdims=True)
        clt = clt + jnp.sum(lt.astype(jnp.float32), axis=1, keepdims=True)
        ceq = ceq + jnp.sum(eq.astype(jnp.float32), axis=1, keepdims=True)
        sall = sall + jnp.sum(jnp.where(eq, vb, 0.0), axis=1, keepdims=True)
        j = (jax.lax.broadcasted_iota(jnp.int32, (tr, ch), 1)
             .astype(jnp.float32) + jnp.float32(c * ch))
        jm = jnp.min(jnp.where(eq, j, BIG), axis=1, keepdims=True)
        vm = jnp.sum(jnp.where(eq & (j == jm), vb, 0.0), axis=1,
                     keepdims=True)
        take = jm < j1
        j1 = jnp.where(take, jm, j1)
        v1 = jnp.where(take, vm, v1)
        return (num, den, clt, ceq, sall, j1, v1)

    z = jnp.zeros((tr, 1), jnp.float32)
    big = jnp.full((tr, 1), BIG, jnp.float32)
    num, den, clt, ceq, sall, j1, v1 = jax.lax.fori_loop(
        0, nch, chunk2, (z, z, z, z, z, big, z))

    need = K - clt
    wstar = 1.0 / (t32 + DELTA)
    tie_num = jnp.where(ceq == need, sall,
                        jnp.where(need == 1.0, v1,
                                  sall * need / jnp.maximum(ceq, 1.0)))
    q = (num + wstar * tie_num) / (den + wstar * need)
    o_ref[...] = jnp.broadcast_to(q, (tr, 128))


def kernel(obs, action, W, b, gamma, beta, keys1, values1, keys2, values2):
    B = obs.shape[0]
    HID = W.shape[1]
    CAP = keys1.shape[0]

    x = jnp.concatenate([obs, action], axis=-1)

    phi = pl.pallas_call(
        _trunk_body,
        out_shape=jax.ShapeDtypeStruct((B, HID), jnp.float32),
    )(x, W, b.reshape(1, HID), gamma.reshape(1, HID), beta.reshape(1, HID))

    BR = min(512, B)
    CB = min(2048, CAP)
    d = pl.pallas_call(
        _dist_body,
        grid=(B // BR, CAP // CB),
        in_specs=[
            pl.BlockSpec((BR, HID), lambda r, c: (r, 0)),
            pl.BlockSpec((CB, HID), lambda r, c: (c, 0)),
        ],
        out_specs=pl.BlockSpec((BR, CB), lambda r, c: (r, c)),
        out_shape=jax.ShapeDtypeStruct((B, CAP), jnp.float32),
    )(phi, keys1)

    TR = min(64, B)
    CH = min(2048, CAP)
    vrow = values1.reshape(1, CAP)
    q = pl.pallas_call(
        functools.partial(_select_body, tr=TR, cap=CAP, ch=CH),
        grid=(B // TR,),
        in_specs=[
            pl.BlockSpec((TR, CAP), lambda r: (r, 0)),
            pl.BlockSpec((1, CAP), lambda r: (0, 0)),
        ],
        out_specs=pl.BlockSpec((TR, 128), lambda r: (r, 0)),
        out_shape=jax.ShapeDtypeStruct((B, 128), jnp.float32),
    )(d, vrow)

    q = q[:, :1]
    return (q, q)


# speedup vs baseline: 1.7149x; 1.0115x over previous
import functools

import jax
import jax.numpy as jnp
from jax.experimental import pallas as pl

K_NEIGHBORS = 32
DELTA = 1e-3
BIG = 3.4e38


def _trunk_body(x_ref, w_ref, b_ref, g_ref, beta_ref, o_ref):
    h = jnp.dot(x_ref[...], w_ref[...], preferred_element_type=jnp.float32)
    h = h + b_ref[...]
    mu = jnp.mean(h, axis=1, keepdims=True)
    var = jnp.mean((h - mu) ** 2, axis=1, keepdims=True)
    hn = (h - mu) / jnp.sqrt(var + 1e-5) * g_ref[...] + beta_ref[...]
    o_ref[...] = jnp.tanh(hn)


def _dist_body(phi_ref, k_ref, o_ref):
    phi = phi_ref[...]
    kb = k_ref[...]
    dot = jax.lax.dot_general(phi, kb, (((1,), (1,)), ((), ())),
                              preferred_element_type=jnp.float32)
    pn = jnp.sum(phi * phi, axis=1, keepdims=True)
    kn = jnp.sum(kb * kb, axis=1)[None, :]
    o_ref[...] = jnp.maximum(pn - 2.0 * dot + kn, 0.0)


def _select_body(d_ref, v_ref, o_ref, *, tr, cap, ch):
    nch = cap // ch
    K = float(K_NEIGHBORS)

    def sweep(_, t):
        def chunk(c, acc):
            m, cgt = acc
            dc = d_ref[:, pl.ds(c * ch, ch)]
            gt = dc > t
            cand = jnp.where(gt, dc, BIG)
            m = jnp.minimum(m, jnp.min(cand, axis=1, keepdims=True))
            cgt = cgt + jnp.sum(gt.astype(jnp.float32), axis=1, keepdims=True)
            return (m, cgt)
        m, cgt = jax.lax.fori_loop(
            0, nch, chunk,
            (jnp.full((tr, 1), BIG, jnp.float32),
             jnp.zeros((tr, 1), jnp.float32)))
        cle = float(cap) - cgt
        return jnp.where(cle >= K, t, m)

    t32 = jax.lax.fori_loop(0, K_NEIGHBORS, sweep,
                            jnp.full((tr, 1), -1.0, jnp.float32))

    def chunk2(c, acc):
        num, den, clt, ceq, sall, j1, v1 = acc
        dc = d_ref[:, pl.ds(c * ch, ch)]
        vb = v_ref[:, pl.ds(c * ch, ch)]
        lt = dc < t32
        eq = dc == t32
        w = jnp.where(lt, 1.0 / (dc + DELTA), 0.0)
        num = num + jnp.sum(w * vb, axis=1, keepdims=True)
        den = den + jnp.sum(w, axis=1, keepdims=True)
        clt = clt + jnp.sum(lt.astype(jnp.float32), axis=1, keepdims=True)
        ceq = ceq + jnp.sum(eq.astype(jnp.float32), axis=1, keepdims=True)
        sall = sall + jnp.sum(jnp.where(eq, vb, 0.0), axis=1, keepdims=True)
        j = (jax.lax.broadcasted_iota(jnp.int32, (tr, ch), 1)
             .astype(jnp.float32) + jnp.float32(c * ch))
        jm = jnp.min(jnp.where(eq, j, BIG), axis=1, keepdims=True)
        vm = jnp.sum(jnp.where(eq & (j == jm), vb, 0.0), axis=1,
                     keepdims=True)
        take = jm < j1
        j1 = jnp.where(take, jm, j1)
        v1 = jnp.where(take, vm, v1)
        return (num, den, clt, ceq, sall, j1, v1)

    z = jnp.zeros((tr, 1), jnp.float32)
    big = jnp.full((tr, 1), BIG, jnp.float32)
    num, den, clt, ceq, sall, j1, v1 = jax.lax.fori_loop(
        0, nch, chunk2, (z, z, z, z, z, big, z))

    need = K - clt
    wstar = 1.0 / (t32 + DELTA)
    tie_num = jnp.where(ceq == need, sall,
                        jnp.where(need == 1.0, v1,
                                  sall * need / jnp.maximum(ceq, 1.0)))
    q = (num + wstar * tie_num) / (den + wstar * need)
    o_ref[...] = jnp.broadcast_to(q, (tr, 128))


def kernel(obs, action, W, b, gamma, beta, keys1, values1, keys2, values2):
    B = obs.shape[0]
    HID = W.shape[1]
    CAP = keys1.shape[0]

    x = jnp.concatenate([obs, action], axis=-1)

    phi = pl.pallas_call(
        _trunk_body,
        out_shape=jax.ShapeDtypeStruct((B, HID), jnp.float32),
    )(x, W, b.reshape(1, HID), gamma.reshape(1, HID), beta.reshape(1, HID))

    BR = min(1024, B)
    CB = min(2048, CAP)
    d = pl.pallas_call(
        _dist_body,
        grid=(B // BR, CAP // CB),
        in_specs=[
            pl.BlockSpec((BR, HID), lambda r, c: (r, 0)),
            pl.BlockSpec((CB, HID), lambda r, c: (c, 0)),
        ],
        out_specs=pl.BlockSpec((BR, CB), lambda r, c: (r, c)),
        out_shape=jax.ShapeDtypeStruct((B, CAP), jnp.float32),
    )(phi, keys1)

    TR = min(64, B)
    CH = min(2048, CAP)
    vrow = values1.reshape(1, CAP)
    q = pl.pallas_call(
        functools.partial(_select_body, tr=TR, cap=CAP, ch=CH),
        grid=(B // TR,),
        in_specs=[
            pl.BlockSpec((TR, CAP), lambda r: (r, 0)),
            pl.BlockSpec((1, CAP), lambda r: (0, 0)),
        ],
        out_specs=pl.BlockSpec((TR, 128), lambda r: (r, 0)),
        out_shape=jax.ShapeDtypeStruct((B, 128), jnp.float32),
    )(d, vrow)

    q = q[:, :1]
    return (q, q)


# speedup vs baseline: 2.1121x; 1.2316x over previous
import functools

import jax
import jax.numpy as jnp
from jax.experimental import pallas as pl

K_NEIGHBORS = 32
DELTA = 1e-3
BIG = 3.4e38


def _trunk_body(x_ref, w_ref, b_ref, g_ref, beta_ref, o_ref):
    h = jnp.dot(x_ref[...], w_ref[...], preferred_element_type=jnp.float32)
    h = h + b_ref[...]
    mu = jnp.mean(h, axis=1, keepdims=True)
    var = jnp.mean((h - mu) ** 2, axis=1, keepdims=True)
    hn = (h - mu) / jnp.sqrt(var + 1e-5) * g_ref[...] + beta_ref[...]
    o_ref[...] = jnp.tanh(hn)


def _dist_body(phi_ref, k_ref, o_ref):
    phi = phi_ref[...]
    kb = k_ref[...]
    dot = jax.lax.dot_general(phi, kb, (((1,), (1,)), ((), ())),
                              preferred_element_type=jnp.float32)
    pn = jnp.sum(phi * phi, axis=1, keepdims=True)
    kn = jnp.sum(kb * kb, axis=1)[None, :]
    o_ref[...] = jnp.maximum(pn - 2.0 * dot + kn, 0.0)


def _select_body(d_ref, v_ref, o_ref, *, tr, cap, ch):
    nch = cap // ch
    K = float(K_NEIGHBORS)

    def sweep(_, t):
        def chunk(c, acc):
            m, cgt = acc
            dc = d_ref[:, pl.ds(c * ch, ch)]
            gt = dc > t
            cand = jnp.where(gt, dc, BIG)
            m = jnp.minimum(m, jnp.min(cand, axis=1, keepdims=True))
            cgt = cgt + jnp.sum(gt.astype(jnp.float32), axis=1, keepdims=True)
            return (m, cgt)
        m, cgt = jax.lax.fori_loop(
            0, nch, chunk,
            (jnp.full((tr, 1), BIG, jnp.float32),
             jnp.zeros((tr, 1), jnp.float32)))
        cle = float(cap) - cgt
        return jnp.where(cle >= K, t, m)

    t32 = jax.lax.fori_loop(0, K_NEIGHBORS, sweep,
                            jnp.full((tr, 1), -1.0, jnp.float32))

    def chunk2(c, acc):
        num, den, clt, ceq, sall, j1, v1 = acc
        dc = d_ref[:, pl.ds(c * ch, ch)]
        vb = v_ref[:, pl.ds(c * ch, ch)]
        lt = dc < t32
        eq = dc == t32
        w = jnp.where(lt, 1.0 / (dc + DELTA), 0.0)
        num = num + jnp.sum(w * vb, axis=1, keepdims=True)
        den = den + jnp.sum(w, axis=1, keepdims=True)
        clt = clt + jnp.sum(lt.astype(jnp.float32), axis=1, keepdims=True)
        ceq = ceq + jnp.sum(eq.astype(jnp.float32), axis=1, keepdims=True)
        sall = sall + jnp.sum(jnp.where(eq, vb, 0.0), axis=1, keepdims=True)
        j = (jax.lax.broadcasted_iota(jnp.int32, (tr, ch), 1)
             .astype(jnp.float32) + jnp.float32(c * ch))
        jm = jnp.min(jnp.where(eq, j, BIG), axis=1, keepdims=True)
        vm = jnp.sum(jnp.where(eq & (j == jm), vb, 0.0), axis=1,
                     keepdims=True)
        take = jm < j1
        j1 = jnp.where(take, jm, j1)
        v1 = jnp.where(take, vm, v1)
        return (num, den, clt, ceq, sall, j1, v1)

    z = jnp.zeros((tr, 1), jnp.float32)
    big = jnp.full((tr, 1), BIG, jnp.float32)
    num, den, clt, ceq, sall, j1, v1 = jax.lax.fori_loop(
        0, nch, chunk2, (z, z, z, z, z, big, z))

    need = K - clt
    wstar = 1.0 / (t32 + DELTA)
    tie_num = jnp.where(ceq == need, sall,
                        jnp.where(need == 1.0, v1,
                                  sall * need / jnp.maximum(ceq, 1.0)))
    q = (num + wstar * tie_num) / (den + wstar * need)
    o_ref[...] = jnp.broadcast_to(q, (tr, 128))


def kernel(obs, action, W, b, gamma, beta, keys1, values1, keys2, values2):
    B = obs.shape[0]
    HID = W.shape[1]
    CAP = keys1.shape[0]

    x = jnp.concatenate([obs, action], axis=-1)

    phi = pl.pallas_call(
        _trunk_body,
        out_shape=jax.ShapeDtypeStruct((B, HID), jnp.float32),
    )(x, W, b.reshape(1, HID), gamma.reshape(1, HID), beta.reshape(1, HID))

    BR = min(1024, B)
    CB = min(2048, CAP)
    d = pl.pallas_call(
        _dist_body,
        grid=(B // BR, CAP // CB),
        in_specs=[
            pl.BlockSpec((BR, HID), lambda r, c: (r, 0)),
            pl.BlockSpec((CB, HID), lambda r, c: (c, 0)),
        ],
        out_specs=pl.BlockSpec((BR, CB), lambda r, c: (r, c)),
        out_shape=jax.ShapeDtypeStruct((B, CAP), jnp.float32),
    )(phi, keys1)

    TR = min(64, B)
    CH = min(4096, CAP)
    vrow = values1.reshape(1, CAP)
    q = pl.pallas_call(
        functools.partial(_select_body, tr=TR, cap=CAP, ch=CH),
        grid=(B // TR,),
        in_specs=[
            pl.BlockSpec((TR, CAP), lambda r: (r, 0)),
            pl.BlockSpec((1, CAP), lambda r: (0, 0)),
        ],
        out_specs=pl.BlockSpec((TR, 128), lambda r: (r, 0)),
        out_shape=jax.ShapeDtypeStruct((B, 128), jnp.float32),
    )(d, vrow)

    q = q[:, :1]
    return (q, q)


# speedup vs baseline: 2.3808x; 1.1272x over previous
import functools

import jax
import jax.numpy as jnp
from jax.experimental import pallas as pl

K_NEIGHBORS = 32
DELTA = 1e-3
BIG = 3.4e38


def _trunk_body(x_ref, w_ref, b_ref, g_ref, beta_ref, o_ref):
    h = jnp.dot(x_ref[...], w_ref[...], preferred_element_type=jnp.float32)
    h = h + b_ref[...]
    mu = jnp.mean(h, axis=1, keepdims=True)
    var = jnp.mean((h - mu) ** 2, axis=1, keepdims=True)
    hn = (h - mu) / jnp.sqrt(var + 1e-5) * g_ref[...] + beta_ref[...]
    o_ref[...] = jnp.tanh(hn)


def _dist_body(phi_ref, k_ref, o_ref):
    phi = phi_ref[...]
    kb = k_ref[...]
    dot = jax.lax.dot_general(phi, kb, (((1,), (1,)), ((), ())),
                              preferred_element_type=jnp.float32)
    pn = jnp.sum(phi * phi, axis=1, keepdims=True)
    kn = jnp.sum(kb * kb, axis=1)[None, :]
    o_ref[...] = jnp.maximum(pn - 2.0 * dot + kn, 0.0)


def _select_body(d_ref, v_ref, o_ref, *, tr, cap, ch):
    nch = cap // ch
    K = float(K_NEIGHBORS)

    def sweep(_, t):
        def chunk(c, acc):
            m, cgt = acc
            dc = d_ref[:, pl.ds(c * ch, ch)]
            gt = dc > t
            cand = jnp.where(gt, dc, BIG)
            m = jnp.minimum(m, jnp.min(cand, axis=1, keepdims=True))
            cgt = cgt + jnp.sum(gt.astype(jnp.float32), axis=1, keepdims=True)
            return (m, cgt)
        m, cgt = jax.lax.fori_loop(
            0, nch, chunk,
            (jnp.full((tr, 1), BIG, jnp.float32),
             jnp.zeros((tr, 1), jnp.float32)))
        cle = float(cap) - cgt
        return jnp.where(cle >= K, t, m)

    t32 = jax.lax.fori_loop(0, K_NEIGHBORS, sweep,
                            jnp.full((tr, 1), -1.0, jnp.float32))

    def chunk2(c, acc):
        num, den, clt, ceq, sall, j1, v1 = acc
        dc = d_ref[:, pl.ds(c * ch, ch)]
        vb = v_ref[:, pl.ds(c * ch, ch)]
        lt = dc < t32
        eq = dc == t32
        w = jnp.where(lt, 1.0 / (dc + DELTA), 0.0)
        num = num + jnp.sum(w * vb, axis=1, keepdims=True)
        den = den + jnp.sum(w, axis=1, keepdims=True)
        clt = clt + jnp.sum(lt.astype(jnp.float32), axis=1, keepdims=True)
        ceq = ceq + jnp.sum(eq.astype(jnp.float32), axis=1, keepdims=True)
        sall = sall + jnp.sum(jnp.where(eq, vb, 0.0), axis=1, keepdims=True)
        j = (jax.lax.broadcasted_iota(jnp.int32, (tr, ch), 1)
             .astype(jnp.float32) + jnp.float32(c * ch))
        jm = jnp.min(jnp.where(eq, j, BIG), axis=1, keepdims=True)
        vm = jnp.sum(jnp.where(eq & (j == jm), vb, 0.0), axis=1,
                     keepdims=True)
        take = jm < j1
        j1 = jnp.where(take, jm, j1)
        v1 = jnp.where(take, vm, v1)
        return (num, den, clt, ceq, sall, j1, v1)

    z = jnp.zeros((tr, 1), jnp.float32)
    big = jnp.full((tr, 1), BIG, jnp.float32)
    num, den, clt, ceq, sall, j1, v1 = jax.lax.fori_loop(
        0, nch, chunk2, (z, z, z, z, z, big, z))

    need = K - clt
    wstar = 1.0 / (t32 + DELTA)
    tie_num = jnp.where(ceq == need, sall,
                        jnp.where(need == 1.0, v1,
                                  sall * need / jnp.maximum(ceq, 1.0)))
    q = (num + wstar * tie_num) / (den + wstar * need)
    o_ref[...] = jnp.broadcast_to(q, (tr, 128))


def kernel(obs, action, W, b, gamma, beta, keys1, values1, keys2, values2):
    B = obs.shape[0]
    HID = W.shape[1]
    CAP = keys1.shape[0]

    x = jnp.concatenate([obs, action], axis=-1)

    phi = pl.pallas_call(
        _trunk_body,
        out_shape=jax.ShapeDtypeStruct((B, HID), jnp.float32),
    )(x, W, b.reshape(1, HID), gamma.reshape(1, HID), beta.reshape(1, HID))

    BR = min(1024, B)
    CB = min(2048, CAP)
    d = pl.pallas_call(
        _dist_body,
        grid=(B // BR, CAP // CB),
        in_specs=[
            pl.BlockSpec((BR, HID), lambda r, c: (r, 0)),
            pl.BlockSpec((CB, HID), lambda r, c: (c, 0)),
        ],
        out_specs=pl.BlockSpec((BR, CB), lambda r, c: (r, c)),
        out_shape=jax.ShapeDtypeStruct((B, CAP), jnp.float32),
    )(phi, keys1)

    TR = min(64, B)
    CH = min(8192, CAP)
    vrow = values1.reshape(1, CAP)
    q = pl.pallas_call(
        functools.partial(_select_body, tr=TR, cap=CAP, ch=CH),
        grid=(B // TR,),
        in_specs=[
            pl.BlockSpec((TR, CAP), lambda r: (r, 0)),
            pl.BlockSpec((1, CAP), lambda r: (0, 0)),
        ],
        out_specs=pl.BlockSpec((TR, 128), lambda r: (r, 0)),
        out_shape=jax.ShapeDtypeStruct((B, 128), jnp.float32),
    )(d, vrow)

    q = q[:, :1]
    return (q, q)


# speedup vs baseline: 2.5332x; 1.0640x over previous
import functools

import jax
import jax.numpy as jnp
from jax.experimental import pallas as pl

K_NEIGHBORS = 32
DELTA = 1e-3
BIG = 3.4e38


def _trunk_body(x_ref, w_ref, b_ref, g_ref, beta_ref, o_ref):
    h = jnp.dot(x_ref[...], w_ref[...], preferred_element_type=jnp.float32)
    h = h + b_ref[...]
    mu = jnp.mean(h, axis=1, keepdims=True)
    var = jnp.mean((h - mu) ** 2, axis=1, keepdims=True)
    hn = (h - mu) / jnp.sqrt(var + 1e-5) * g_ref[...] + beta_ref[...]
    o_ref[...] = jnp.tanh(hn)


def _dist_body(phi_ref, k_ref, o_ref):
    phi = phi_ref[...]
    kb = k_ref[...]
    dot = jax.lax.dot_general(phi, kb, (((1,), (1,)), ((), ())),
                              preferred_element_type=jnp.float32)
    pn = jnp.sum(phi * phi, axis=1, keepdims=True)
    kn = jnp.sum(kb * kb, axis=1)[None, :]
    o_ref[...] = jnp.maximum(pn - 2.0 * dot + kn, 0.0)


def _select_body(d_ref, v_ref, o_ref, *, tr, cap, ch):
    nch = cap // ch
    K = float(K_NEIGHBORS)

    def sweep(_, t):
        def chunk(c, acc):
            m, cgt = acc
            dc = d_ref[:, pl.ds(c * ch, ch)]
            gt = dc > t
            cand = jnp.where(gt, dc, BIG)
            m = jnp.minimum(m, jnp.min(cand, axis=1, keepdims=True))
            cgt = cgt + jnp.sum(gt.astype(jnp.float32), axis=1, keepdims=True)
            return (m, cgt)
        m, cgt = jax.lax.fori_loop(
            0, nch, chunk,
            (jnp.full((tr, 1), BIG, jnp.float32),
             jnp.zeros((tr, 1), jnp.float32)))
        cle = float(cap) - cgt
        return jnp.where(cle >= K, t, m)

    t32 = jax.lax.fori_loop(0, K_NEIGHBORS, sweep,
                            jnp.full((tr, 1), -1.0, jnp.float32))

    def chunk2(c, acc):
        num, den, clt, ceq, sall, j1, v1 = acc
        dc = d_ref[:, pl.ds(c * ch, ch)]
        vb = v_ref[:, pl.ds(c * ch, ch)]
        lt = dc < t32
        eq = dc == t32
        w = jnp.where(lt, 1.0 / (dc + DELTA), 0.0)
        num = num + jnp.sum(w * vb, axis=1, keepdims=True)
        den = den + jnp.sum(w, axis=1, keepdims=True)
        clt = clt + jnp.sum(lt.astype(jnp.float32), axis=1, keepdims=True)
        ceq = ceq + jnp.sum(eq.astype(jnp.float32), axis=1, keepdims=True)
        sall = sall + jnp.sum(jnp.where(eq, vb, 0.0), axis=1, keepdims=True)
        j = (jax.lax.broadcasted_iota(jnp.int32, (tr, ch), 1)
             .astype(jnp.float32) + jnp.float32(c * ch))
        jm = jnp.min(jnp.where(eq, j, BIG), axis=1, keepdims=True)
        vm = jnp.sum(jnp.where(eq & (j == jm), vb, 0.0), axis=1,
                     keepdims=True)
        take = jm < j1
        j1 = jnp.where(take, jm, j1)
        v1 = jnp.where(take, vm, v1)
        return (num, den, clt, ceq, sall, j1, v1)

    z = jnp.zeros((tr, 1), jnp.float32)
    big = jnp.full((tr, 1), BIG, jnp.float32)
    num, den, clt, ceq, sall, j1, v1 = jax.lax.fori_loop(
        0, nch, chunk2, (z, z, z, z, z, big, z))

    need = K - clt
    wstar = 1.0 / (t32 + DELTA)
    tie_num = jnp.where(ceq == need, sall,
                        jnp.where(need == 1.0, v1,
                                  sall * need / jnp.maximum(ceq, 1.0)))
    q = (num + wstar * tie_num) / (den + wstar * need)
    o_ref[...] = jnp.broadcast_to(q, (tr, 128))


def kernel(obs, action, W, b, gamma, beta, keys1, values1, keys2, values2):
    B = obs.shape[0]
    HID = W.shape[1]
    CAP = keys1.shape[0]

    x = jnp.concatenate([obs, action], axis=-1)

    phi = pl.pallas_call(
        _trunk_body,
        out_shape=jax.ShapeDtypeStruct((B, HID), jnp.float32),
    )(x, W, b.reshape(1, HID), gamma.reshape(1, HID), beta.reshape(1, HID))

    BR = min(1024, B)
    CB = min(2048, CAP)
    d = pl.pallas_call(
        _dist_body,
        grid=(B // BR, CAP // CB),
        in_specs=[
            pl.BlockSpec((BR, HID), lambda r, c: (r, 0)),
            pl.BlockSpec((CB, HID), lambda r, c: (c, 0)),
        ],
        out_specs=pl.BlockSpec((BR, CB), lambda r, c: (r, c)),
        out_shape=jax.ShapeDtypeStruct((B, CAP), jnp.float32),
    )(phi, keys1)

    TR = min(64, B)
    CH = min(16384, CAP)
    vrow = values1.reshape(1, CAP)
    q = pl.pallas_call(
        functools.partial(_select_body, tr=TR, cap=CAP, ch=CH),
        grid=(B // TR,),
        in_specs=[
            pl.BlockSpec((TR, CAP), lambda r: (r, 0)),
            pl.BlockSpec((1, CAP), lambda r: (0, 0)),
        ],
        out_specs=pl.BlockSpec((TR, 128), lambda r: (r, 0)),
        out_shape=jax.ShapeDtypeStruct((B, 128), jnp.float32),
    )(d, vrow)

    q = q[:, :1]
    return (q, q)


# speedup vs baseline: 2.6126x; 1.0313x over previous
import functools

import jax
import jax.numpy as jnp
from jax.experimental import pallas as pl

K_NEIGHBORS = 32
DELTA = 1e-3
BIG = 3.4e38


def _trunk_body(x_ref, w_ref, b_ref, g_ref, beta_ref, o_ref):
    h = jnp.dot(x_ref[...], w_ref[...], preferred_element_type=jnp.float32)
    h = h + b_ref[...]
    mu = jnp.mean(h, axis=1, keepdims=True)
    var = jnp.mean((h - mu) ** 2, axis=1, keepdims=True)
    hn = (h - mu) / jnp.sqrt(var + 1e-5) * g_ref[...] + beta_ref[...]
    o_ref[...] = jnp.tanh(hn)


def _dist_body(phi_ref, k_ref, o_ref):
    phi = phi_ref[...]
    kb = k_ref[...]
    dot = jax.lax.dot_general(phi, kb, (((1,), (1,)), ((), ())),
                              preferred_element_type=jnp.float32)
    pn = jnp.sum(phi * phi, axis=1, keepdims=True)
    kn = jnp.sum(kb * kb, axis=1)[None, :]
    o_ref[...] = jnp.maximum(pn - 2.0 * dot + kn, 0.0)


def _select_body(d_ref, v_ref, o_ref, *, tr, cap, ch):
    nch = cap // ch
    K = float(K_NEIGHBORS)

    def sweep(_, t):
        def chunk(c, acc):
            m, cgt = acc
            dc = d_ref[:, pl.ds(c * ch, ch)]
            gt = dc > t
            cand = jnp.where(gt, dc, BIG)
            m = jnp.minimum(m, jnp.min(cand, axis=1, keepdims=True))
            cgt = cgt + jnp.sum(gt.astype(jnp.float32), axis=1, keepdims=True)
            return (m, cgt)
        m, cgt = jax.lax.fori_loop(
            0, nch, chunk,
            (jnp.full((tr, 1), BIG, jnp.float32),
             jnp.zeros((tr, 1), jnp.float32)))
        cle = float(cap) - cgt
        return jnp.where(cle >= K, t, m)

    t32 = jax.lax.fori_loop(0, K_NEIGHBORS, sweep,
                            jnp.full((tr, 1), -1.0, jnp.float32))

    def chunk2(c, acc):
        num, den, clt, ceq, sall, j1, v1 = acc
        dc = d_ref[:, pl.ds(c * ch, ch)]
        vb = v_ref[:, pl.ds(c * ch, ch)]
        lt = dc < t32
        eq = dc == t32
        w = jnp.where(lt, 1.0 / (dc + DELTA), 0.0)
        num = num + jnp.sum(w * vb, axis=1, keepdims=True)
        den = den + jnp.sum(w, axis=1, keepdims=True)
        clt = clt + jnp.sum(lt.astype(jnp.float32), axis=1, keepdims=True)
        ceq = ceq + jnp.sum(eq.astype(jnp.float32), axis=1, keepdims=True)
        sall = sall + jnp.sum(jnp.where(eq, vb, 0.0), axis=1, keepdims=True)
        j = (jax.lax.broadcasted_iota(jnp.int32, (tr, ch), 1)
             .astype(jnp.float32) + jnp.float32(c * ch))
        jm = jnp.min(jnp.where(eq, j, BIG), axis=1, keepdims=True)
        vm = jnp.sum(jnp.where(eq & (j == jm), vb, 0.0), axis=1,
                     keepdims=True)
        take = jm < j1
        j1 = jnp.where(take, jm, j1)
        v1 = jnp.where(take, vm, v1)
        return (num, den, clt, ceq, sall, j1, v1)

    z = jnp.zeros((tr, 1), jnp.float32)
    big = jnp.full((tr, 1), BIG, jnp.float32)
    num, den, clt, ceq, sall, j1, v1 = jax.lax.fori_loop(
        0, nch, chunk2, (z, z, z, z, z, big, z))

    need = K - clt
    wstar = 1.0 / (t32 + DELTA)
    tie_num = jnp.where(ceq == need, sall,
                        jnp.where(need == 1.0, v1,
                                  sall * need / jnp.maximum(ceq, 1.0)))
    q = (num + wstar * tie_num) / (den + wstar * need)
    o_ref[...] = jnp.broadcast_to(q, (tr, 128))


def kernel(obs, action, W, b, gamma, beta, keys1, values1, keys2, values2):
    B = obs.shape[0]
    HID = W.shape[1]
    CAP = keys1.shape[0]

    x = jnp.concatenate([obs, action], axis=-1)

    phi = pl.pallas_call(
        _trunk_body,
        out_shape=jax.ShapeDtypeStruct((B, HID), jnp.float32),
    )(x, W, b.reshape(1, HID), gamma.reshape(1, HID), beta.reshape(1, HID))

    BR = min(1024, B)
    CB = min(2048, CAP)
    d = pl.pallas_call(
        _dist_body,
        grid=(B // BR, CAP // CB),
        in_specs=[
            pl.BlockSpec((BR, HID), lambda r, c: (r, 0)),
            pl.BlockSpec((CB, HID), lambda r, c: (c, 0)),
        ],
        out_specs=pl.BlockSpec((BR, CB), lambda r, c: (r, c)),
        out_shape=jax.ShapeDtypeStruct((B, CAP), jnp.float32),
    )(phi, keys1)

    TR = min(64, B)
    CH = min(32768, CAP)
    vrow = values1.reshape(1, CAP)
    q = pl.pallas_call(
        functools.partial(_select_body, tr=TR, cap=CAP, ch=CH),
        grid=(B // TR,),
        in_specs=[
            pl.BlockSpec((TR, CAP), lambda r: (r, 0)),
            pl.BlockSpec((1, CAP), lambda r: (0, 0)),
        ],
        out_specs=pl.BlockSpec((TR, 128), lambda r: (r, 0)),
        out_shape=jax.ShapeDtypeStruct((B, 128), jnp.float32),
    )(d, vrow)

    q = q[:, :1]
    return (q, q)
